# Initial kernel scaffold; baseline (speedup 1.0000x reference)
import jax
import jax.numpy as jnp
from jax.experimental import pallas as pl


def kernel(scores, gumbel_noise, absorbed_mask, k_per_row):
    raise NotImplementedError("write your pallas kernel here")


# speedup vs baseline: 4.4758x; 4.4758x over previous
import jax
import jax.numpy as jnp
from jax.experimental import pallas as pl

_B = 128
_N = 4096
_ROWS = 16
_GRID = _B // _ROWS
def _body(scores_ref, gumbel_ref, absorbed_ref, k_ref, out_ref):
    _INT_MIN = jnp.int32(-(2**31))
    scores = scores_ref[...]
    g = gumbel_ref[...]
    absorbed = absorbed_ref[...] != 0
    k_in = k_ref[...]

    gi = jax.lax.bitcast_convert_type(g, jnp.int32)
    ordk = jnp.where(gi < 0, gi ^ jnp.int32(0x7FFFFFFF), gi)
    ordk = jnp.where(g == 0.0, jnp.int32(0), ordk)
    key = jnp.where(absorbed, ordk, _INT_MIN)

    num_abs = jnp.sum(absorbed.astype(jnp.int32), axis=1, keepdims=True)
    k = jnp.minimum(num_abs, k_in)

    prefix_u = jnp.zeros_like(k)
    for b in range(31, -1, -1):
        bit = _INT_MIN if b == 31 else jnp.int32(1 << b)
        cand_u = prefix_u | bit
        cand_s = cand_u ^ _INT_MIN
        cnt = jnp.sum((key >= cand_s).astype(jnp.int32), axis=1, keepdims=True)
        prefix_u = jnp.where(cnt >= k, cand_u, prefix_u)
    t = prefix_u ^ _INT_MIN

    mask_gt = key > t
    cnt_gt = jnp.sum(mask_gt.astype(jnp.int32), axis=1, keepdims=True)
    ties_needed = k - cnt_gt
    mask_eq = key == t

    idx = jax.lax.broadcasted_iota(jnp.int32, (_ROWS, _N), 1)
    cut = jnp.zeros_like(k)
    for b in range(11, -1, -1):
        cand = cut + jnp.int32((1 << b) - 1)
        cnt = jnp.sum((mask_eq & (idx <= cand)).astype(jnp.int32), axis=1,
                      keepdims=True)
        cut = jnp.where(cnt >= ties_needed, cut, cut + jnp.int32(1 << b))

    to_unmask = (mask_gt | (mask_eq & (idx <= cut))) & (k > 0)

    m = jnp.max(scores, axis=1, keepdims=True)
    e = jnp.exp(scores - m)
    s = jnp.sum(e, axis=1, keepdims=True)
    out_ref[...] = jnp.where(to_unmask, e / s, 0.0)


def kernel(scores, gumbel_noise, absorbed_mask, k_per_row):
    k2 = k_per_row.astype(jnp.int32).reshape(_B, 1)
    return pl.pallas_call(
        _body,
        grid=(_GRID,),
        in_specs=[
            pl.BlockSpec((_ROWS, _N), lambda i: (i, 0)),
            pl.BlockSpec((_ROWS, _N), lambda i: (i, 0)),
            pl.BlockSpec((_ROWS, _N), lambda i: (i, 0)),
            pl.BlockSpec((_ROWS, 1), lambda i: (i, 0)),
        ],
        out_specs=pl.BlockSpec((_ROWS, _N), lambda i: (i, 0)),
        out_shape=jax.ShapeDtypeStruct((_B, _N), jnp.float32),
    )(scores, gumbel_noise, absorbed_mask, k2)
